# Initial kernel scaffold; baseline (speedup 1.0000x reference)
import jax
import jax.numpy as jnp
from jax.experimental import pallas as pl


def kernel(x, edge_index, W0, b0, W1, b1):
    raise NotImplementedError("write your pallas kernel here")


# speedup vs baseline: 20.6825x; 20.6825x over previous
import functools

import jax
import jax.numpy as jnp
from jax import lax
from jax.experimental import pallas as pl
from jax.experimental.pallas import tpu as pltpu
from jax.experimental.pallas import tpu_sc as plsc

N = 10000
D = 128
E = 320000

NC = 2
NS = 16
NW = NC * NS
CHUNK = 128
CPW = 80
E_PAD = NW * CPW * CHUNK
N_PAD = 10240
RPS = N_PAD // NS
BM = 512
BM_OUT = 400

_mesh = plsc.VectorSubcoreMesh(core_axis_name="c", subcore_axis_name="s")


@functools.partial(
    pl.kernel,
    out_type=jax.ShapeDtypeStruct((NC, N_PAD, 16), jnp.float32),
    mesh=_mesh,
    scratch_types=[
        pltpu.VMEM((CHUNK, 16), jnp.float32),
        pltpu.VMEM((CPW, CHUNK), jnp.int32),
        pltpu.VMEM_SHARED((N_PAD, 16), jnp.float32),
    ],
)
def _deg_partials(dst_hbm, ones_hbm, z16_hbm, out_hbm, ones_v, idx_v, acc_sh):
    c = lax.axis_index("c")
    s = lax.axis_index("s")
    w = s * NC + c
    pltpu.sync_copy(ones_hbm, ones_v)
    pltpu.sync_copy(z16_hbm.at[pl.ds(s * RPS, RPS)],
                    acc_sh.at[pl.ds(s * RPS, RPS)])
    pltpu.sync_copy(dst_hbm.at[pl.ds(w * CPW, CPW)], idx_v)
    plsc.subcore_barrier()

    @pl.loop(0, CPW)
    def _(j):
        pltpu.sync_copy(ones_v, acc_sh.at[idx_v.at[j]], add=True)

    plsc.subcore_barrier()
    pltpu.sync_copy(acc_sh.at[pl.ds(s * RPS, RPS)],
                    out_hbm.at[c].at[pl.ds(s * RPS, RPS)])


@functools.partial(
    pl.kernel,
    out_type=jax.ShapeDtypeStruct((NC, N_PAD, D), jnp.float32),
    mesh=_mesh,
    scratch_types=[
        pltpu.VMEM((CPW, CHUNK), jnp.int32),
        pltpu.VMEM((CPW, CHUNK), jnp.int32),
        pltpu.VMEM((CHUNK, D), jnp.float32),
        pltpu.VMEM_SHARED((N_PAD, D), jnp.float32),
    ],
)
def _edge_partials(g_hbm, src_hbm, dst_hbm, z_hbm, out_hbm,
                   src_v, dst_v, rows_v, acc_sh):
    c = lax.axis_index("c")
    s = lax.axis_index("s")
    w = s * NC + c
    pltpu.sync_copy(z_hbm.at[pl.ds(s * RPS, RPS)],
                    acc_sh.at[pl.ds(s * RPS, RPS)])
    pltpu.sync_copy(src_hbm.at[pl.ds(w * CPW, CPW)], src_v)
    pltpu.sync_copy(dst_hbm.at[pl.ds(w * CPW, CPW)], dst_v)
    plsc.subcore_barrier()

    @pl.loop(0, CPW)
    def _(j):
        pltpu.sync_copy(g_hbm.at[src_v.at[j]], rows_v)
        pltpu.sync_copy(rows_v, acc_sh.at[dst_v.at[j]], add=True)

    plsc.subcore_barrier()
    pltpu.sync_copy(acc_sh.at[pl.ds(s * RPS, RPS)],
                    out_hbm.at[c].at[pl.ds(s * RPS, RPS)])


def _mm_body(x_ref, w_ref, o_ref):
    o_ref[...] = jnp.dot(x_ref[...], w_ref[...],
                         preferred_element_type=jnp.float32)


_matmul = pl.pallas_call(
    _mm_body,
    grid=(N_PAD // BM,),
    in_specs=[
        pl.BlockSpec((BM, D), lambda i: (i, 0)),
        pl.BlockSpec((D, D), lambda i: (0, 0)),
    ],
    out_specs=pl.BlockSpec((BM, D), lambda i: (i, 0)),
    out_shape=jax.ShapeDtypeStruct((N_PAD, D), jnp.float32),
)


def _prep_body(degp_ref, h_ref, g_ref, dinv_ref):
    deg = degp_ref[0, :, 0:1] + degp_ref[1, :, 0:1] + 1.0
    dinv = lax.rsqrt(deg)
    dinv_ref[...] = dinv
    g_ref[...] = h_ref[...] * dinv


_prep = pl.pallas_call(
    _prep_body,
    grid=(N_PAD // BM,),
    in_specs=[
        pl.BlockSpec((NC, BM, 16), lambda i: (0, i, 0)),
        pl.BlockSpec((BM, D), lambda i: (i, 0)),
    ],
    out_specs=[
        pl.BlockSpec((BM, D), lambda i: (i, 0)),
        pl.BlockSpec((BM, 1), lambda i: (i, 0)),
    ],
    out_shape=[
        jax.ShapeDtypeStruct((N_PAD, D), jnp.float32),
        jax.ShapeDtypeStruct((N_PAD, 1), jnp.float32),
    ],
)


def _layer_body(p_ref, g_ref, dinv_ref, b_ref, w_ref, o_ref):
    dinv = dinv_ref[...]
    h = jnp.maximum(
        dinv * (p_ref[0] + p_ref[1] + g_ref[...]) + b_ref[...], 0.0)
    o_ref[...] = jnp.dot(h, w_ref[...],
                         preferred_element_type=jnp.float32) * dinv


_layer = pl.pallas_call(
    _layer_body,
    grid=(N_PAD // BM,),
    in_specs=[
        pl.BlockSpec((NC, BM, D), lambda i: (0, i, 0)),
        pl.BlockSpec((BM, D), lambda i: (i, 0)),
        pl.BlockSpec((BM, 1), lambda i: (i, 0)),
        pl.BlockSpec((1, D), lambda i: (0, 0)),
        pl.BlockSpec((D, D), lambda i: (0, 0)),
    ],
    out_specs=pl.BlockSpec((BM, D), lambda i: (i, 0)),
    out_shape=jax.ShapeDtypeStruct((N_PAD, D), jnp.float32),
)


def _final_body(p_ref, g_ref, dinv_ref, b_ref, o_ref):
    o_ref[...] = jnp.maximum(
        dinv_ref[...] * (p_ref[0] + p_ref[1] + g_ref[...]) + b_ref[...], 0.0)


_final = pl.pallas_call(
    _final_body,
    grid=(N // BM_OUT,),
    in_specs=[
        pl.BlockSpec((NC, BM_OUT, D), lambda i: (0, i, 0)),
        pl.BlockSpec((BM_OUT, D), lambda i: (i, 0)),
        pl.BlockSpec((BM_OUT, 1), lambda i: (i, 0)),
        pl.BlockSpec((1, D), lambda i: (0, 0)),
    ],
    out_specs=pl.BlockSpec((BM_OUT, D), lambda i: (i, 0)),
    out_shape=jax.ShapeDtypeStruct((N, D), jnp.float32),
)


def kernel(x, edge_index, W0, b0, W1, b1):
    src = edge_index[0].astype(jnp.int32)
    dst = edge_index[1].astype(jnp.int32)
    pad = N + (jnp.arange(E_PAD - E, dtype=jnp.int32) % (N_PAD - N))
    src_p = jnp.concatenate([src, pad]).reshape(E_PAD // CHUNK, CHUNK)
    dst_p = jnp.concatenate([dst, pad]).reshape(E_PAD // CHUNK, CHUNK)

    x_pad = jnp.zeros((N_PAD, D), jnp.float32).at[:N].set(x)
    zeros = jnp.zeros((N_PAD, D), jnp.float32)
    z16 = jnp.zeros((N_PAD, 16), jnp.float32)
    ones = jnp.ones((CHUNK, 16), jnp.float32)

    degp = _deg_partials(dst_p, ones, z16)
    h0 = _matmul(x_pad, W0)
    g0, dinv = _prep(degp, h0)
    p1 = _edge_partials(g0, src_p, dst_p, zeros)
    g1 = _layer(p1, g0, dinv, b0.reshape(1, D), W1)
    p2 = _edge_partials(g1, src_p, dst_p, zeros)
    return _final(p2, g1, dinv, b1.reshape(1, D))


# speedup vs baseline: 28.5610x; 1.3809x over previous
import functools

import jax
import jax.numpy as jnp
from jax import lax
from jax.experimental import pallas as pl
from jax.experimental.pallas import tpu as pltpu
from jax.experimental.pallas import tpu_sc as plsc

N = 10000
D = 128
E = 320000

NC = 2
NS = 16
NW = NC * NS
CHUNK = 128
CPW = 80
E_PAD = NW * CPW * CHUNK
N_PAD = 10240
RPS = N_PAD // NS
BM = 512
BM_OUT = 400

_mesh = plsc.VectorSubcoreMesh(core_axis_name="c", subcore_axis_name="s")


@functools.partial(
    pl.kernel,
    out_type=jax.ShapeDtypeStruct((NC, N_PAD, 16), jnp.float32),
    mesh=_mesh,
    scratch_types=[
        pltpu.VMEM((CHUNK, 16), jnp.float32),
        pltpu.VMEM((CPW, CHUNK), jnp.int32),
        pltpu.VMEM_SHARED((N_PAD, 16), jnp.float32),
        pltpu.SemaphoreType.DMA,
    ],
)
def _deg_partials(dst_hbm, ones_hbm, z16_hbm, out_hbm, ones_v, idx_v, acc_sh,
                  sem):
    c = lax.axis_index("c")
    s = lax.axis_index("s")
    w = s * NC + c
    pltpu.sync_copy(ones_hbm, ones_v)
    pltpu.sync_copy(z16_hbm.at[pl.ds(s * RPS, RPS)],
                    acc_sh.at[pl.ds(s * RPS, RPS)])
    pltpu.sync_copy(dst_hbm.at[pl.ds(w * CPW, CPW)], idx_v)
    plsc.subcore_barrier()

    @pl.loop(0, CPW)
    def _(j):
        pltpu.sync_copy(ones_v, acc_sh.at[idx_v.at[j]], add=True)

    plsc.subcore_barrier()
    pltpu.sync_copy(acc_sh.at[pl.ds(s * RPS, RPS)],
                    out_hbm.at[c].at[pl.ds(s * RPS, RPS)])


@functools.partial(
    pl.kernel,
    out_type=jax.ShapeDtypeStruct((NC, N_PAD, D), jnp.float32),
    mesh=_mesh,
    scratch_types=[
        pltpu.VMEM((CPW // 2, CHUNK), jnp.int32),
        pltpu.VMEM((CPW // 2, CHUNK), jnp.int32),
        pltpu.VMEM((2, CHUNK, D), jnp.float32),
        pltpu.VMEM_SHARED((N_PAD, D), jnp.float32),
        pltpu.SemaphoreType.DMA,
        pltpu.SemaphoreType.DMA,
    ],
)
def _edge_partials(g_hbm, src_hbm, dst_hbm, z_hbm, out_hbm,
                   src_v, dst_v, rows_v, acc_sh, sem0, sem1):
    c = lax.axis_index("c")
    s = lax.axis_index("s")
    w = s * NC + c
    H = CPW // 2
    pltpu.sync_copy(z_hbm.at[pl.ds(s * RPS, RPS)],
                    acc_sh.at[pl.ds(s * RPS, RPS)])
    plsc.subcore_barrier()

    for h in range(2):
        pltpu.sync_copy(src_hbm.at[pl.ds(w * CPW + h * H, H)], src_v)
        pltpu.sync_copy(dst_hbm.at[pl.ds(w * CPW + h * H, H)], dst_v)

        pltpu.async_copy(g_hbm.at[src_v.at[0]], rows_v.at[0], sem0)
        pltpu.async_copy(g_hbm.at[src_v.at[1]], rows_v.at[1], sem1)

        @pl.loop(0, H - 2, step=2)
        def _(j):
            pltpu.make_async_copy(g_hbm.at[src_v.at[j]], rows_v.at[0],
                                  sem0).wait()
            pltpu.sync_copy(rows_v.at[0], acc_sh.at[dst_v.at[j]], add=True)
            pltpu.async_copy(g_hbm.at[src_v.at[j + 2]], rows_v.at[0], sem0)

            pltpu.make_async_copy(g_hbm.at[src_v.at[j + 1]], rows_v.at[1],
                                  sem1).wait()
            pltpu.sync_copy(rows_v.at[1], acc_sh.at[dst_v.at[j + 1]], add=True)
            pltpu.async_copy(g_hbm.at[src_v.at[j + 3]], rows_v.at[1], sem1)

        pltpu.make_async_copy(g_hbm.at[src_v.at[H - 2]], rows_v.at[0],
                              sem0).wait()
        pltpu.sync_copy(rows_v.at[0], acc_sh.at[dst_v.at[H - 2]], add=True)
        pltpu.make_async_copy(g_hbm.at[src_v.at[H - 1]], rows_v.at[1],
                              sem1).wait()
        pltpu.sync_copy(rows_v.at[1], acc_sh.at[dst_v.at[H - 1]], add=True)

    plsc.subcore_barrier()
    pltpu.sync_copy(acc_sh.at[pl.ds(s * RPS, RPS)],
                    out_hbm.at[c].at[pl.ds(s * RPS, RPS)])


def _mm_body(x_ref, w_ref, o_ref):
    o_ref[...] = jnp.dot(x_ref[...], w_ref[...],
                         preferred_element_type=jnp.float32)


_matmul = pl.pallas_call(
    _mm_body,
    grid=(N_PAD // BM,),
    in_specs=[
        pl.BlockSpec((BM, D), lambda i: (i, 0)),
        pl.BlockSpec((D, D), lambda i: (0, 0)),
    ],
    out_specs=pl.BlockSpec((BM, D), lambda i: (i, 0)),
    out_shape=jax.ShapeDtypeStruct((N_PAD, D), jnp.float32),
)


def _prep_body(degp_ref, h_ref, g_ref, dinv_ref):
    deg = degp_ref[0, :, 0:1] + degp_ref[1, :, 0:1] + 1.0
    dinv = lax.rsqrt(deg)
    dinv_ref[...] = dinv
    g_ref[...] = h_ref[...] * dinv


_prep = pl.pallas_call(
    _prep_body,
    grid=(N_PAD // BM,),
    in_specs=[
        pl.BlockSpec((NC, BM, 16), lambda i: (0, i, 0)),
        pl.BlockSpec((BM, D), lambda i: (i, 0)),
    ],
    out_specs=[
        pl.BlockSpec((BM, D), lambda i: (i, 0)),
        pl.BlockSpec((BM, 1), lambda i: (i, 0)),
    ],
    out_shape=[
        jax.ShapeDtypeStruct((N_PAD, D), jnp.float32),
        jax.ShapeDtypeStruct((N_PAD, 1), jnp.float32),
    ],
)


def _layer_body(p_ref, g_ref, dinv_ref, b_ref, w_ref, o_ref):
    dinv = dinv_ref[...]
    h = jnp.maximum(
        dinv * (p_ref[0] + p_ref[1] + g_ref[...]) + b_ref[...], 0.0)
    o_ref[...] = jnp.dot(h, w_ref[...],
                         preferred_element_type=jnp.float32) * dinv


_layer = pl.pallas_call(
    _layer_body,
    grid=(N_PAD // BM,),
    in_specs=[
        pl.BlockSpec((NC, BM, D), lambda i: (0, i, 0)),
        pl.BlockSpec((BM, D), lambda i: (i, 0)),
        pl.BlockSpec((BM, 1), lambda i: (i, 0)),
        pl.BlockSpec((1, D), lambda i: (0, 0)),
        pl.BlockSpec((D, D), lambda i: (0, 0)),
    ],
    out_specs=pl.BlockSpec((BM, D), lambda i: (i, 0)),
    out_shape=jax.ShapeDtypeStruct((N_PAD, D), jnp.float32),
)


def _final_body(p_ref, g_ref, dinv_ref, b_ref, o_ref):
    o_ref[...] = jnp.maximum(
        dinv_ref[...] * (p_ref[0] + p_ref[1] + g_ref[...]) + b_ref[...], 0.0)


_final = pl.pallas_call(
    _final_body,
    grid=(N // BM_OUT,),
    in_specs=[
        pl.BlockSpec((NC, BM_OUT, D), lambda i: (0, i, 0)),
        pl.BlockSpec((BM_OUT, D), lambda i: (i, 0)),
        pl.BlockSpec((BM_OUT, 1), lambda i: (i, 0)),
        pl.BlockSpec((1, D), lambda i: (0, 0)),
    ],
    out_specs=pl.BlockSpec((BM_OUT, D), lambda i: (i, 0)),
    out_shape=jax.ShapeDtypeStruct((N, D), jnp.float32),
)


def kernel(x, edge_index, W0, b0, W1, b1):
    src = edge_index[0].astype(jnp.int32)
    dst = edge_index[1].astype(jnp.int32)
    pad = N + (jnp.arange(E_PAD - E, dtype=jnp.int32) % (N_PAD - N))
    src_p = jnp.concatenate([src, pad]).reshape(E_PAD // CHUNK, CHUNK)
    dst_p = jnp.concatenate([dst, pad]).reshape(E_PAD // CHUNK, CHUNK)

    x_pad = jnp.zeros((N_PAD, D), jnp.float32).at[:N].set(x)
    zeros = jnp.zeros((N_PAD, D), jnp.float32)
    z16 = jnp.zeros((N_PAD, 16), jnp.float32)
    ones = jnp.ones((CHUNK, 16), jnp.float32)

    degp = _deg_partials(dst_p, ones, z16)
    h0 = _matmul(x_pad, W0)
    g0, dinv = _prep(degp, h0)
    p1 = _edge_partials(g0, src_p, dst_p, zeros)
    g1 = _layer(p1, g0, dinv, b0.reshape(1, D), W1)
    p2 = _edge_partials(g1, src_p, dst_p, zeros)
    return _final(p2, g1, dinv, b1.reshape(1, D))
